# baseline (speedup 1.0000x reference)
import functools

import jax
import jax.numpy as jnp
from jax import lax
from jax.experimental import pallas as pl
from jax.experimental.pallas import tpu as pltpu
from jax.experimental.pallas import tpu_sc as plsc

_USER_NUM = 1000000
_NC = 2
_NS = 16
_NW = _NC * _NS
_L = 16
_CHUNK = 128


def _ffm_body(user_hbm, item_hbm, fc_hbm, bias_hbm, emb0_hbm, emb1_hbm,
              out_hbm, u_idx, i_idx, rows_u, rows_i, fc_u, fc_i, bias_v,
              out_v, sem_u, sem_i, sem_g, b_per_w):
    wid = lax.axis_index("s") * _NC + lax.axis_index("c")
    base = wid * b_per_w
    n_chunks = b_per_w // _CHUNK
    n_groups = b_per_w // _L

    cp_u = pltpu.async_copy(user_hbm.at[pl.ds(base, b_per_w)], u_idx, sem_u)
    cp_i = pltpu.async_copy(item_hbm.at[pl.ds(base, b_per_w)], i_idx, sem_i)
    pltpu.sync_copy(bias_hbm, bias_v)

    cp_u.wait()
    gathers = []
    for j in range(n_chunks):
        sl = pl.ds(j * _CHUNK, _CHUNK)
        gathers.append(pltpu.async_copy(emb1_hbm.at[u_idx.at[sl]],
                                        rows_u.at[sl], sem_g))
        gathers.append(pltpu.async_copy(fc_hbm.at[u_idx.at[sl]],
                                        fc_u.at[sl], sem_g))

    cp_i.wait()
    for v in range(b_per_w // _L):
        sl = pl.ds(v * _L, _L)
        i_idx[sl] = i_idx[sl] + _USER_NUM
    for j in range(n_chunks):
        sl = pl.ds(j * _CHUNK, _CHUNK)
        gathers.append(pltpu.async_copy(emb0_hbm.at[i_idx.at[sl]],
                                        rows_i.at[sl], sem_g))
        gathers.append(pltpu.async_copy(fc_hbm.at[i_idx.at[sl]],
                                        fc_i.at[sl], sem_g))
    for g in gathers:
        g.wait()

    iota = lax.iota(jnp.int32, _L)
    bias_bc = bias_v[...]

    def group(g, _):
        rid = g * _L + iota
        acc = plsc.load_gather(fc_u, [rid]) + plsc.load_gather(fc_i, [rid])
        acc = acc + bias_bc
        for k in range(_L):
            kv = jnp.full((_L,), k, jnp.int32)
            a = plsc.load_gather(rows_u, [rid, kv])
            b = plsc.load_gather(rows_i, [rid, kv])
            acc = acc + a * b
        out_v[pl.ds(g * _L, _L)] = acc
        return _

    lax.fori_loop(0, n_groups, group, None)
    pltpu.sync_copy(out_v, out_hbm.at[pl.ds(base, b_per_w)])


def kernel(user, item, features, fc, bias, emb0, emb1):
    del features
    b = user.shape[0]
    b_per_w = b // _NW
    mesh = plsc.VectorSubcoreMesh(core_axis_name="c", subcore_axis_name="s")
    run = pl.kernel(
        functools.partial(_ffm_body, b_per_w=b_per_w),
        out_type=jax.ShapeDtypeStruct((b,), jnp.float32),
        mesh=mesh,
        scratch_types=[
            pltpu.VMEM((b_per_w,), jnp.int32),
            pltpu.VMEM((b_per_w,), jnp.int32),
            pltpu.VMEM((b_per_w, _L), jnp.float32),
            pltpu.VMEM((b_per_w, _L), jnp.float32),
            pltpu.VMEM((b_per_w,), jnp.float32),
            pltpu.VMEM((b_per_w,), jnp.float32),
            pltpu.VMEM((_L,), jnp.float32),
            pltpu.VMEM((b_per_w,), jnp.float32),
            pltpu.SemaphoreType.DMA,
            pltpu.SemaphoreType.DMA,
            pltpu.SemaphoreType.DMA,
        ],
        compiler_params=pltpu.CompilerParams(
            needs_layout_passes=False, use_tc_tiling_on_sc=False),
    )
    bias16 = jnp.broadcast_to(bias, (_L,))
    return run(user, item, fc.reshape(-1), bias16, emb0, emb1)
